# baseline (speedup 1.0000x reference)
import functools
import jax
import jax.numpy as jnp
from jax import lax
from jax.experimental import pallas as pl
from jax.experimental.pallas import tpu as pltpu
from jax.experimental.pallas import tpu_sc as plsc

_NC = 2
_NS = 16
_NW = _NC * _NS
_B = 16384
_TOT = 2 * _B
_PER_W = _TOT // _NW
_HALF_W = _PER_W // 2
_CHUNK = 128
_NCHUNK = _PER_W // _CHUNK
_D = 64
_PAIR = 2 * _D
_MARGIN = 4.0


def _rsqrt16(x):
    i = plsc.bitcast(x, jnp.int32)
    i = 0x5F3759DF - lax.shift_right_logical(i, 1)
    y = plsc.bitcast(i, jnp.float32)
    for _ in range(3):
        y = y * (1.5 - 0.5 * x * y * y)
    return y


def _sc_call(ent2, rel2, idx_blocks):
    mesh = plsc.VectorSubcoreMesh(
        core_axis_name="c", subcore_axis_name="s",
        num_cores=_NC, num_subcores=_NS)

    @functools.partial(
        pl.kernel,
        out_type=(
            jax.ShapeDtypeStruct((_TOT,), jnp.float32),
            jax.ShapeDtypeStruct((_NW, 16), jnp.float32),
        ),
        mesh=mesh,
        compiler_params=pltpu.CompilerParams(
            needs_layout_passes=False, use_tc_tiling_on_sc=True),
        scratch_types=[
            pltpu.VMEM((5, _CHUNK), jnp.int32),
            pltpu.VMEM((5, _CHUNK), jnp.int32),
            pltpu.VMEM((_CHUNK, _PAIR), jnp.float32),
            pltpu.VMEM((_CHUNK, _PAIR), jnp.float32),
            pltpu.VMEM((_CHUNK, _PAIR), jnp.float32),
            pltpu.VMEM((_CHUNK, _PAIR), jnp.float32),
            pltpu.VMEM((_CHUNK, _PAIR), jnp.float32),
            pltpu.VMEM((_PER_W,), jnp.float32),
            pltpu.VMEM((16,), jnp.float32),
            pltpu.SemaphoreType.DMA,
        ],
    )
    def trans_e(ent_hbm, rel_hbm, idx_hbm, scores_hbm, part_hbm,
                idx_v, hx_v, h_v, r_v, t_v, qr_v, qe_v, sc_v, par_v, sem):
        w = lax.axis_index("s") * _NC + lax.axis_index("c")
        lane = jnp.arange(16, dtype=jnp.int32)
        zero = jnp.zeros((16,), jnp.float32)

        def chunk_body(c, carry):
            pltpu.sync_copy(idx_hbm.at[w, c], idx_v)

            for col in range(5):
                for seg in range(_CHUNK // 16):
                    v = idx_v[col, pl.ds(seg * 16, 16)]
                    hx_v[col, pl.ds(seg * 16, 16)] = (
                        lax.shift_right_logical(v, 1))

            cps = [
                pltpu.async_copy(ent_hbm.at[hx_v.at[0]], h_v, sem),
                pltpu.async_copy(rel_hbm.at[hx_v.at[1]], r_v, sem),
                pltpu.async_copy(ent_hbm.at[hx_v.at[2]], t_v, sem),
                pltpu.async_copy(rel_hbm.at[hx_v.at[3]], qr_v, sem),
                pltpu.async_copy(ent_hbm.at[hx_v.at[4]], qe_v, sem),
            ]
            for cp in cps:
                cp.wait()

            def group_body(g, gcarry):
                rows = g * 16 + lane
                offs = []
                for col in range(5):
                    p = idx_v[col, pl.ds(g * 16, 16)]
                    offs.append(lax.shift_left(p & 1, 6))
                off_h, off_r, off_t, off_qr, off_qe = offs

                def norm_body(d, acc):
                    sh, st, sq = acc
                    dv = jnp.full((16,), d, jnp.int32)
                    hv = plsc.load_gather(h_v, [rows, off_h + dv])
                    tv = plsc.load_gather(t_v, [rows, off_t + dv])
                    qv = plsc.load_gather(qe_v, [rows, off_qe + dv])
                    return (sh + hv * hv, st + tv * tv, sq + qv * qv)

                sh, st, sq = lax.fori_loop(
                    0, _D, norm_body, (zero, zero, zero), unroll=8)
                inv_h = _rsqrt16(sh)
                inv_t = _rsqrt16(st)
                inv_q = _rsqrt16(sq)

                def comb_body(d, acc):
                    dv = jnp.full((16,), d, jnp.int32)
                    hv = plsc.load_gather(h_v, [rows, off_h + dv])
                    rv = plsc.load_gather(r_v, [rows, off_r + dv])
                    tv = plsc.load_gather(t_v, [rows, off_t + dv])
                    qrv = plsc.load_gather(qr_v, [rows, off_qr + dv])
                    qev = plsc.load_gather(qe_v, [rows, off_qe + dv])
                    s = hv * inv_h + rv - tv * inv_t + qrv - qev * inv_q
                    return acc + jnp.abs(s)

                dist = lax.fori_loop(0, _D, comb_body, zero, unroll=8)
                sc_v[pl.ds(c * _CHUNK + g * 16, 16)] = dist
                return gcarry

            lax.fori_loop(0, _CHUNK // 16, group_body, 0)
            return carry

        lax.fori_loop(0, _NCHUNK, chunk_body, 0)

        def loss_body(i, p):
            pv = sc_v[pl.ds(i * 16, 16)]
            nv = sc_v[pl.ds(_HALF_W + i * 16, 16)]
            return p + jnp.maximum(pv - nv + _MARGIN, 0.0)

        par_v[...] = lax.fori_loop(0, _HALF_W // 16, loss_body, zero,
                                   unroll=4)
        pltpu.sync_copy(sc_v.at[pl.ds(0, _HALF_W)],
                        scores_hbm.at[pl.ds(w * _HALF_W, _HALF_W)])
        pltpu.sync_copy(sc_v.at[pl.ds(_HALF_W, _HALF_W)],
                        scores_hbm.at[pl.ds(_B + w * _HALF_W, _HALF_W)])
        pltpu.sync_copy(par_v, part_hbm.at[w])

    return trans_e(ent2, rel2, idx_blocks)


def kernel(entity_emb, relation_emb, batch_positives, batch_negatives):
    ent2 = entity_emb.reshape(entity_emb.shape[0] // 2, _PAIR)
    rel2 = relation_emb.reshape(relation_emb.shape[0] // 2, _PAIR)
    pos = batch_positives.reshape(_NW, _NCHUNK // 2, _CHUNK, 5)
    neg = batch_negatives.reshape(_NW, _NCHUNK // 2, _CHUNK, 5)
    idx_blocks = jnp.concatenate(
        [pos.transpose(0, 1, 3, 2), neg.transpose(0, 1, 3, 2)], axis=1)
    scores, partials = _sc_call(ent2, rel2, idx_blocks)
    loss = jnp.sum(partials) / _B
    return scores[:_B], scores[_B:], loss


# speedup vs baseline: 1.1969x; 1.1969x over previous
import functools
import jax
import jax.numpy as jnp
from jax import lax
from jax.experimental import pallas as pl
from jax.experimental.pallas import tpu as pltpu
from jax.experimental.pallas import tpu_sc as plsc

_NC = 2
_NS = 16
_NW = _NC * _NS
_B = 16384
_TOT = 2 * _B
_PER_W = _TOT // _NW
_HALF_W = _PER_W // 2
_CHUNK = 64
_NCHUNK = _PER_W // _CHUNK
_NGRP = _CHUNK // 16
_D = 64
_PAIR = 2 * _D
_MARGIN = 4.0


def _rsqrt16(x):
    i = plsc.bitcast(x, jnp.int32)
    i = 0x5F3759DF - lax.shift_right_logical(i, 1)
    y = plsc.bitcast(i, jnp.float32)
    for _ in range(3):
        y = y * (1.5 - 0.5 * x * y * y)
    return y


def _sc_call(ent2, rel2, idx_blocks):
    mesh = plsc.VectorSubcoreMesh(
        core_axis_name="c", subcore_axis_name="s",
        num_cores=_NC, num_subcores=_NS)

    @functools.partial(
        pl.kernel,
        out_type=(
            jax.ShapeDtypeStruct((_TOT,), jnp.float32),
            jax.ShapeDtypeStruct((_NW, 16), jnp.float32),
        ),
        mesh=mesh,
        compiler_params=pltpu.CompilerParams(
            needs_layout_passes=False, use_tc_tiling_on_sc=True),
        scratch_types=[
            [pltpu.VMEM((5, _CHUNK), jnp.int32) for _ in range(2)],
            [pltpu.VMEM((5, _CHUNK), jnp.int32) for _ in range(2)],
            [[pltpu.VMEM((_CHUNK, _PAIR), jnp.float32) for _ in range(5)]
             for _ in range(2)],
            pltpu.VMEM((_PER_W,), jnp.float32),
            pltpu.VMEM((16,), jnp.float32),
            [pltpu.SemaphoreType.DMA for _ in range(2)],
        ],
    )
    def trans_e(ent_hbm, rel_hbm, idx_hbm, scores_hbm, part_hbm,
                idx_v, hx_v, rows_v, sc_v, par_v, sems):
        w = lax.axis_index("s") * _NC + lax.axis_index("c")
        lane = jnp.arange(16, dtype=jnp.int32)
        zero = jnp.zeros((16,), jnp.float32)

        def issue(c, p):
            pltpu.sync_copy(idx_hbm.at[w, c], idx_v[p])
            for col in range(5):
                for seg in range(_CHUNK // 16):
                    v = idx_v[p][col, pl.ds(seg * 16, 16)]
                    hx_v[p][col, pl.ds(seg * 16, 16)] = (
                        lax.shift_right_logical(v, 1))
            tabs = (ent_hbm, rel_hbm, ent_hbm, rel_hbm, ent_hbm)
            for col in range(5):
                pltpu.async_copy(tabs[col].at[hx_v[p].at[col]],
                                 rows_v[p][col], sems[p])

        def drain(p):
            for col in range(5):
                pltpu.make_async_copy(
                    (ent_hbm if col % 2 == 0 else rel_hbm).at[
                        hx_v[p].at[col]],
                    rows_v[p][col], sems[p]).wait()

        def compute(c, p):
            h_v, r_v, t_v, qr_v, qe_v = rows_v[p]

            def group_body(g, gcarry):
                rows = g * 16 + lane
                offs = []
                for col in range(5):
                    par = idx_v[p][col, pl.ds(g * 16, 16)]
                    offs.append(lax.shift_left(par & 1, 6))
                off_h, off_r, off_t, off_qr, off_qe = offs

                def norm_body(d, acc):
                    sh, st, sq = acc
                    dv = (d + lane) & 63
                    hv = plsc.load_gather(h_v, [rows, off_h + dv])
                    tv = plsc.load_gather(t_v, [rows, off_t + dv])
                    qv = plsc.load_gather(qe_v, [rows, off_qe + dv])
                    return (sh + hv * hv, st + tv * tv, sq + qv * qv)

                sh, st, sq = lax.fori_loop(
                    0, _D, norm_body, (zero, zero, zero), unroll=8)
                inv_h = _rsqrt16(sh)
                inv_t = _rsqrt16(st)
                inv_q = _rsqrt16(sq)

                def comb_body(d, acc):
                    dv = (d + lane) & 63
                    hv = plsc.load_gather(h_v, [rows, off_h + dv])
                    rv = plsc.load_gather(r_v, [rows, off_r + dv])
                    tv = plsc.load_gather(t_v, [rows, off_t + dv])
                    qrv = plsc.load_gather(qr_v, [rows, off_qr + dv])
                    qev = plsc.load_gather(qe_v, [rows, off_qe + dv])
                    s = hv * inv_h + rv - tv * inv_t + qrv - qev * inv_q
                    return acc + jnp.abs(s)

                dist = lax.fori_loop(0, _D, comb_body, zero, unroll=8)
                sc_v[pl.ds(c * _CHUNK + g * 16, 16)] = dist
                return gcarry

            lax.fori_loop(0, _NGRP, group_body, 0)

        issue(0, 0)

        def pair_body(i, carry):
            c0 = 2 * i
            drain(0)
            issue(c0 + 1, 1)
            compute(c0, 0)
            drain(1)
            issue(lax.min(c0 + 2, _NCHUNK - 1), 0)
            compute(c0 + 1, 1)
            return carry

        lax.fori_loop(0, _NCHUNK // 2, pair_body, 0)
        drain(0)

        def loss_body(i, pacc):
            pv = sc_v[pl.ds(i * 16, 16)]
            nv = sc_v[pl.ds(_HALF_W + i * 16, 16)]
            return pacc + jnp.maximum(pv - nv + _MARGIN, 0.0)

        par_v[...] = lax.fori_loop(0, _HALF_W // 16, loss_body, zero,
                                   unroll=4)
        pltpu.sync_copy(sc_v.at[pl.ds(0, _HALF_W)],
                        scores_hbm.at[pl.ds(w * _HALF_W, _HALF_W)])
        pltpu.sync_copy(sc_v.at[pl.ds(_HALF_W, _HALF_W)],
                        scores_hbm.at[pl.ds(_B + w * _HALF_W, _HALF_W)])
        pltpu.sync_copy(par_v, part_hbm.at[w])

    return trans_e(ent2, rel2, idx_blocks)


def kernel(entity_emb, relation_emb, batch_positives, batch_negatives):
    ent2 = entity_emb.reshape(entity_emb.shape[0] // 2, _PAIR)
    rel2 = relation_emb.reshape(relation_emb.shape[0] // 2, _PAIR)
    pos = batch_positives.reshape(_NW, _NCHUNK // 2, _CHUNK, 5)
    neg = batch_negatives.reshape(_NW, _NCHUNK // 2, _CHUNK, 5)
    idx_blocks = jnp.concatenate(
        [pos.transpose(0, 1, 3, 2), neg.transpose(0, 1, 3, 2)], axis=1)
    scores, partials = _sc_call(ent2, rel2, idx_blocks)
    loss = jnp.sum(partials) / _B
    return scores[:_B], scores[_B:], loss


# speedup vs baseline: 1.7382x; 1.4522x over previous
import functools
import jax
import jax.numpy as jnp
from jax import lax
from jax.experimental import pallas as pl
from jax.experimental.pallas import tpu as pltpu
from jax.experimental.pallas import tpu_sc as plsc

_NC = 2
_NS = 16
_NW = _NC * _NS
_B = 16384
_TOT = 2 * _B
_PER_W = _TOT // _NW
_HALF_W = _PER_W // 2
_CHUNK = 64
_NCHUNK = _PER_W // _CHUNK
_NGRP = _CHUNK // 16
_D = 64
_PAIR = 2 * _D
_MARGIN = 4.0

_N = 1000000
_E = 2048
_TGRID = (_N + 2 * _E - 1) // (2 * _E)
_NBLK = (_N + _E - 1) // _E
_ROWS = _TGRID * _E


def _pack_table(table, normalize):
    def body(x1_ref, x2_ref, o_ref):
        xl = jnp.swapaxes(x1_ref[...], 0, 1)
        xh = jnp.swapaxes(x2_ref[...], 0, 1)
        x = jnp.concatenate([xl, xh], axis=1)
        if normalize:
            sl = jnp.sum(xl * xl, axis=1, keepdims=True)
            sh = jnp.sum(xh * xh, axis=1, keepdims=True)
            inv = lax.rsqrt(jnp.concatenate(
                [jnp.broadcast_to(sl, xl.shape),
                 jnp.broadcast_to(sh, xh.shape)], axis=1))
            x = x * inv
        o_ref[...] = x

    fn = pl.pallas_call(
        body,
        grid=(_TGRID,),
        in_specs=[
            pl.BlockSpec((64, _E), lambda i: (0, 2 * i)),
            pl.BlockSpec((64, _E),
                         lambda i: (0, jnp.minimum(2 * i + 1, _NBLK - 1))),
        ],
        out_specs=pl.BlockSpec((_E, _PAIR), lambda i: (i, 0)),
        out_shape=jax.ShapeDtypeStruct((_ROWS, _PAIR), jnp.float32),
    )
    tv = table.T
    return fn(tv, tv)


def _sc_call(ent2, rel2, idx_blocks):
    mesh = plsc.VectorSubcoreMesh(
        core_axis_name="c", subcore_axis_name="s",
        num_cores=_NC, num_subcores=_NS)

    @functools.partial(
        pl.kernel,
        out_type=(
            jax.ShapeDtypeStruct((_TOT,), jnp.float32),
            jax.ShapeDtypeStruct((_NW, 16), jnp.float32),
        ),
        mesh=mesh,
        compiler_params=pltpu.CompilerParams(
            needs_layout_passes=False, use_tc_tiling_on_sc=True),
        scratch_types=[
            [pltpu.VMEM((5, _CHUNK), jnp.int32) for _ in range(2)],
            [pltpu.VMEM((5, _CHUNK), jnp.int32) for _ in range(2)],
            [[pltpu.VMEM((_CHUNK, _PAIR), jnp.float32) for _ in range(5)]
             for _ in range(2)],
            pltpu.VMEM((_PER_W,), jnp.float32),
            pltpu.VMEM((16,), jnp.float32),
            [pltpu.SemaphoreType.DMA for _ in range(2)],
        ],
    )
    def trans_e(ent_hbm, rel_hbm, idx_hbm, scores_hbm, part_hbm,
                idx_v, hx_v, rows_v, sc_v, par_v, sems):
        w = lax.axis_index("s") * _NC + lax.axis_index("c")
        lane = jnp.arange(16, dtype=jnp.int32)
        zero = jnp.zeros((16,), jnp.float32)

        def issue(c, p):
            pltpu.sync_copy(idx_hbm.at[w, c], idx_v[p])
            for col in range(5):
                for seg in range(_CHUNK // 16):
                    v = idx_v[p][col, pl.ds(seg * 16, 16)]
                    row = lax.shift_left(
                        lax.shift_right_logical(v, 12), 11) | (v & 2047)
                    hx_v[p][col, pl.ds(seg * 16, 16)] = row
            tabs = (ent_hbm, rel_hbm, ent_hbm, rel_hbm, ent_hbm)
            for col in range(5):
                pltpu.async_copy(tabs[col].at[hx_v[p].at[col]],
                                 rows_v[p][col], sems[p])

        def drain(p):
            for col in range(5):
                pltpu.make_async_copy(
                    (ent_hbm if col % 2 == 0 else rel_hbm).at[
                        hx_v[p].at[col]],
                    rows_v[p][col], sems[p]).wait()

        def compute(c, p):
            h_v, r_v, t_v, qr_v, qe_v = rows_v[p]

            def group_body(g, gcarry):
                rows = g * 16 + lane
                offs = []
                for col in range(5):
                    v = idx_v[p][col, pl.ds(g * 16, 16)]
                    offs.append(lax.shift_left(
                        lax.shift_right_logical(v, 11) & 1, 6))
                off_h, off_r, off_t, off_qr, off_qe = offs

                def comb_body(d, acc):
                    dv = (d + lane) & 63
                    hv = plsc.load_gather(h_v, [rows, off_h + dv])
                    rv = plsc.load_gather(r_v, [rows, off_r + dv])
                    tv = plsc.load_gather(t_v, [rows, off_t + dv])
                    qrv = plsc.load_gather(qr_v, [rows, off_qr + dv])
                    qev = plsc.load_gather(qe_v, [rows, off_qe + dv])
                    s = hv + rv - tv + qrv - qev
                    return acc + jnp.abs(s)

                dist = lax.fori_loop(0, _D, comb_body, zero, unroll=8)
                sc_v[pl.ds(c * _CHUNK + g * 16, 16)] = dist
                return gcarry

            lax.fori_loop(0, _NGRP, group_body, 0)

        issue(0, 0)

        def pair_body(i, carry):
            c0 = 2 * i
            drain(0)
            issue(c0 + 1, 1)
            compute(c0, 0)
            drain(1)
            issue(lax.min(c0 + 2, _NCHUNK - 1), 0)
            compute(c0 + 1, 1)
            return carry

        lax.fori_loop(0, _NCHUNK // 2, pair_body, 0)
        drain(0)

        def loss_body(i, pacc):
            pv = sc_v[pl.ds(i * 16, 16)]
            nv = sc_v[pl.ds(_HALF_W + i * 16, 16)]
            return pacc + jnp.maximum(pv - nv + _MARGIN, 0.0)

        par_v[...] = lax.fori_loop(0, _HALF_W // 16, loss_body, zero,
                                   unroll=4)
        pltpu.sync_copy(sc_v.at[pl.ds(0, _HALF_W)],
                        scores_hbm.at[pl.ds(w * _HALF_W, _HALF_W)])
        pltpu.sync_copy(sc_v.at[pl.ds(_HALF_W, _HALF_W)],
                        scores_hbm.at[pl.ds(_B + w * _HALF_W, _HALF_W)])
        pltpu.sync_copy(par_v, part_hbm.at[w])

    return trans_e(ent2, rel2, idx_blocks)


def kernel(entity_emb, relation_emb, batch_positives, batch_negatives):
    ent2 = _pack_table(entity_emb, normalize=True)
    rel2 = _pack_table(relation_emb, normalize=False)
    pos = batch_positives.reshape(_NW, _NCHUNK // 2, _CHUNK, 5)
    neg = batch_negatives.reshape(_NW, _NCHUNK // 2, _CHUNK, 5)
    idx_blocks = jnp.concatenate(
        [pos.transpose(0, 1, 3, 2), neg.transpose(0, 1, 3, 2)], axis=1)
    scores, partials = _sc_call(ent2, rel2, idx_blocks)
    loss = jnp.sum(partials) / _B
    return scores[:_B], scores[_B:], loss


# speedup vs baseline: 3.3533x; 1.9292x over previous
import functools
import jax
import jax.numpy as jnp
from jax import lax
from jax.experimental import pallas as pl
from jax.experimental.pallas import tpu as pltpu
from jax.experimental.pallas import tpu_sc as plsc

_NC = 2
_NS = 16
_NW = _NC * _NS
_B = 16384
_TOT = 2 * _B
_PER_W = _TOT // _NW
_HALF_W = _PER_W // 2
_CHUNK = 64
_NCHUNK = _PER_W // _CHUNK
_NGRP = _CHUNK // 16
_D = 64
_PAIR = 2 * _D
_MARGIN = 4.0

_N = 1000000
_E = 8192
_LE = _E.bit_length() - 1
_TGRID = (_N + 2 * _E - 1) // (2 * _E)
_NBLK = (_N + _E - 1) // _E
_ROWS = _TGRID * _E


def _pack_table(table, normalize):
    def body(x1_ref, x2_ref, o_ref):
        x1 = x1_ref[...]
        x2 = x2_ref[...]
        if normalize:
            x1 = x1 * lax.rsqrt(
                jnp.sum(x1 * x1, axis=0, keepdims=True))
            x2 = x2 * lax.rsqrt(
                jnp.sum(x2 * x2, axis=0, keepdims=True))
        xc = jnp.concatenate([x1, x2], axis=0)
        o_ref[...] = jnp.swapaxes(xc, 0, 1)

    fn = pl.pallas_call(
        body,
        grid=(_TGRID,),
        in_specs=[
            pl.BlockSpec((64, _E), lambda i: (0, 2 * i)),
            pl.BlockSpec((64, _E),
                         lambda i: (0, jnp.minimum(2 * i + 1, _NBLK - 1))),
        ],
        out_specs=pl.BlockSpec((_E, _PAIR), lambda i: (i, 0)),
        out_shape=jax.ShapeDtypeStruct((_ROWS, _PAIR), jnp.float32),
    )
    tv = table.T
    return fn(tv, tv)


def _sc_call(ent2, rel2, idx_blocks):
    mesh = plsc.VectorSubcoreMesh(
        core_axis_name="c", subcore_axis_name="s",
        num_cores=_NC, num_subcores=_NS)

    @functools.partial(
        pl.kernel,
        out_type=(
            jax.ShapeDtypeStruct((_TOT,), jnp.float32),
            jax.ShapeDtypeStruct((_NW, 16), jnp.float32),
        ),
        mesh=mesh,
        compiler_params=pltpu.CompilerParams(
            needs_layout_passes=False, use_tc_tiling_on_sc=True),
        scratch_types=[
            [pltpu.VMEM((5, _CHUNK), jnp.int32) for _ in range(2)],
            [pltpu.VMEM((5, _CHUNK), jnp.int32) for _ in range(2)],
            [[pltpu.VMEM((_CHUNK, _PAIR), jnp.float32) for _ in range(5)]
             for _ in range(2)],
            pltpu.VMEM((_PER_W,), jnp.float32),
            pltpu.VMEM((16,), jnp.float32),
            [pltpu.SemaphoreType.DMA for _ in range(2)],
        ],
    )
    def trans_e(ent_hbm, rel_hbm, idx_hbm, scores_hbm, part_hbm,
                idx_v, hx_v, rows_v, sc_v, par_v, sems):
        w = lax.axis_index("s") * _NC + lax.axis_index("c")
        lane = jnp.arange(16, dtype=jnp.int32)
        zero = jnp.zeros((16,), jnp.float32)

        def issue(c, p):
            pltpu.sync_copy(idx_hbm.at[w, c], idx_v[p])
            for col in range(5):
                for seg in range(_CHUNK // 16):
                    v = idx_v[p][col, pl.ds(seg * 16, 16)]
                    row = lax.shift_left(
                        lax.shift_right_logical(v, _LE + 1), _LE) | (
                            v & (_E - 1))
                    hx_v[p][col, pl.ds(seg * 16, 16)] = row
            tabs = (ent_hbm, rel_hbm, ent_hbm, rel_hbm, ent_hbm)
            for col in range(5):
                pltpu.async_copy(tabs[col].at[hx_v[p].at[col]],
                                 rows_v[p][col], sems[p])

        def drain(p):
            for col in range(5):
                pltpu.make_async_copy(
                    (ent_hbm if col % 2 == 0 else rel_hbm).at[
                        hx_v[p].at[col]],
                    rows_v[p][col], sems[p]).wait()

        def compute(c, p):
            h_v, r_v, t_v, qr_v, qe_v = rows_v[p]

            def group_body(g, gcarry):
                rows = g * 16 + lane
                offs = []
                for col in range(5):
                    v = idx_v[p][col, pl.ds(g * 16, 16)]
                    offs.append(lax.shift_left(
                        lax.shift_right_logical(v, _LE) & 1, 6))
                off_h, off_r, off_t, off_qr, off_qe = offs

                def comb_body(d, acc):
                    dv = (d + lane) & 63
                    hv = plsc.load_gather(h_v, [rows, off_h + dv])
                    rv = plsc.load_gather(r_v, [rows, off_r + dv])
                    tv = plsc.load_gather(t_v, [rows, off_t + dv])
                    qrv = plsc.load_gather(qr_v, [rows, off_qr + dv])
                    qev = plsc.load_gather(qe_v, [rows, off_qe + dv])
                    s = hv + rv - tv + qrv - qev
                    return acc + jnp.abs(s)

                dist = lax.fori_loop(0, _D, comb_body, zero, unroll=8)
                sc_v[pl.ds(c * _CHUNK + g * 16, 16)] = dist
                return gcarry

            lax.fori_loop(0, _NGRP, group_body, 0)

        issue(0, 0)

        def pair_body(i, carry):
            c0 = 2 * i
            drain(0)
            issue(c0 + 1, 1)
            compute(c0, 0)
            drain(1)
            issue(lax.min(c0 + 2, _NCHUNK - 1), 0)
            compute(c0 + 1, 1)
            return carry

        lax.fori_loop(0, _NCHUNK // 2, pair_body, 0)
        drain(0)

        def loss_body(i, pacc):
            pv = sc_v[pl.ds(i * 16, 16)]
            nv = sc_v[pl.ds(_HALF_W + i * 16, 16)]
            return pacc + jnp.maximum(pv - nv + _MARGIN, 0.0)

        par_v[...] = lax.fori_loop(0, _HALF_W // 16, loss_body, zero,
                                   unroll=4)
        pltpu.sync_copy(sc_v.at[pl.ds(0, _HALF_W)],
                        scores_hbm.at[pl.ds(w * _HALF_W, _HALF_W)])
        pltpu.sync_copy(sc_v.at[pl.ds(_HALF_W, _HALF_W)],
                        scores_hbm.at[pl.ds(_B + w * _HALF_W, _HALF_W)])
        pltpu.sync_copy(par_v, part_hbm.at[w])

    return trans_e(ent2, rel2, idx_blocks)


def kernel(entity_emb, relation_emb, batch_positives, batch_negatives):
    ent2 = _pack_table(entity_emb, normalize=True)
    rel2 = _pack_table(relation_emb, normalize=False)
    pos = batch_positives.reshape(_NW, _NCHUNK // 2, _CHUNK, 5)
    neg = batch_negatives.reshape(_NW, _NCHUNK // 2, _CHUNK, 5)
    idx_blocks = jnp.concatenate(
        [pos.transpose(0, 1, 3, 2), neg.transpose(0, 1, 3, 2)], axis=1)
    scores, partials = _sc_call(ent2, rel2, idx_blocks)
    loss = jnp.sum(partials) / _B
    return scores[:_B], scores[_B:], loss


# speedup vs baseline: 4.1298x; 1.2316x over previous
import functools
import jax
import jax.numpy as jnp
from jax import lax
from jax.experimental import pallas as pl
from jax.experimental.pallas import tpu as pltpu
from jax.experimental.pallas import tpu_sc as plsc

_NC = 2
_NS = 16
_NW = _NC * _NS
_B = 16384
_TOT = 2 * _B
_PER_W = _TOT // _NW
_HALF_W = _PER_W // 2
_CHUNK = 64
_NCHUNK = _PER_W // _CHUNK
_NGRP = _CHUNK // 16
_D = 64
_PAIR = 2 * _D
_MARGIN = 4.0

_N = 1000000
_E = 8192
_LE = _E.bit_length() - 1
_TGRID = (_N + 4 * _E - 1) // (4 * _E)
_NBLK = (_N + _E - 1) // _E
_ROWS = _TGRID * _E


def _pack_table(table, normalize):
    def pack32(x):
        u = lax.bitcast_convert_type(x, jnp.uint32)
        rnd = jnp.uint32(0x7FFF) + (
            lax.shift_right_logical(u, jnp.uint32(16)) & jnp.uint32(1))
        u = u + rnd
        hi = u[:32, :] & jnp.uint32(0xFFFF0000)
        lo = lax.shift_right_logical(u[32:, :], jnp.uint32(16))
        return lax.bitcast_convert_type(hi | lo, jnp.int32)

    def body(x1_ref, x2_ref, x3_ref, x4_ref, o_ref):
        ws = []
        for ref in (x1_ref, x2_ref, x3_ref, x4_ref):
            x = ref[...]
            if normalize:
                x = x * lax.rsqrt(
                    jnp.sum(x * x, axis=0, keepdims=True))
            ws.append(pack32(x))
        xc = jnp.concatenate(ws, axis=0)
        o_ref[...] = jnp.swapaxes(xc, 0, 1)

    def imap(j):
        return lambda i: (0, jnp.minimum(4 * i + j, _NBLK - 1))

    fn = pl.pallas_call(
        body,
        grid=(_TGRID,),
        in_specs=[pl.BlockSpec((64, _E), imap(j)) for j in range(4)],
        out_specs=pl.BlockSpec((_E, _PAIR), lambda i: (i, 0)),
        out_shape=jax.ShapeDtypeStruct((_ROWS, _PAIR), jnp.int32),
    )
    tv = table.T
    return fn(tv, tv, tv, tv)


def _sc_call(ent2, rel2, idx_blocks):
    mesh = plsc.VectorSubcoreMesh(
        core_axis_name="c", subcore_axis_name="s",
        num_cores=_NC, num_subcores=_NS)

    @functools.partial(
        pl.kernel,
        out_type=(
            jax.ShapeDtypeStruct((_TOT,), jnp.float32),
            jax.ShapeDtypeStruct((_NW, 16), jnp.float32),
        ),
        mesh=mesh,
        compiler_params=pltpu.CompilerParams(
            needs_layout_passes=False, use_tc_tiling_on_sc=True),
        scratch_types=[
            [pltpu.VMEM((5, _CHUNK), jnp.int32) for _ in range(2)],
            [pltpu.VMEM((5, _CHUNK), jnp.int32) for _ in range(2)],
            [[pltpu.VMEM((_CHUNK, _PAIR), jnp.int32) for _ in range(5)]
             for _ in range(2)],
            pltpu.VMEM((_PER_W,), jnp.float32),
            pltpu.VMEM((16,), jnp.float32),
            [pltpu.SemaphoreType.DMA for _ in range(2)],
        ],
    )
    def trans_e(ent_hbm, rel_hbm, idx_hbm, scores_hbm, part_hbm,
                idx_v, hx_v, rows_v, sc_v, par_v, sems):
        w = lax.axis_index("s") * _NC + lax.axis_index("c")
        lane = jnp.arange(16, dtype=jnp.int32)
        zero = jnp.zeros((16,), jnp.float32)

        def issue(c, p):
            pltpu.sync_copy(idx_hbm.at[w, c], idx_v[p])
            for col in range(5):
                for seg in range(_CHUNK // 16):
                    v = idx_v[p][col, pl.ds(seg * 16, 16)]
                    row = lax.shift_left(
                        lax.shift_right_logical(v, _LE + 2), _LE) | (
                            v & (_E - 1))
                    hx_v[p][col, pl.ds(seg * 16, 16)] = row
            tabs = (ent_hbm, rel_hbm, ent_hbm, rel_hbm, ent_hbm)
            for col in range(5):
                pltpu.async_copy(tabs[col].at[hx_v[p].at[col]],
                                 rows_v[p][col], sems[p])

        def drain(p):
            for col in range(5):
                pltpu.make_async_copy(
                    (ent_hbm if col % 2 == 0 else rel_hbm).at[
                        hx_v[p].at[col]],
                    rows_v[p][col], sems[p]).wait()

        def compute(c, p):
            h_v, r_v, t_v, qr_v, qe_v = rows_v[p]

            def group_body(g, gcarry):
                rows = g * 16 + lane
                offs = []
                for col in range(5):
                    v = idx_v[p][col, pl.ds(g * 16, 16)]
                    offs.append(lax.shift_left(
                        lax.shift_right_logical(v, _LE) & 3, 5))
                off_h, off_r, off_t, off_qr, off_qe = offs

                mhi = jnp.full((16,), 0xFFFF0000, jnp.uint32)

                def unpk(w):
                    u = plsc.bitcast(w, jnp.uint32)
                    hi = plsc.bitcast(u & mhi, jnp.float32)
                    lo = plsc.bitcast(
                        lax.shift_left(u, jnp.uint32(16)), jnp.float32)
                    return hi, lo

                def comb_body(d, acc):
                    dv = (d + lane) & 31
                    h1, h2 = unpk(plsc.load_gather(h_v, [rows, off_h + dv]))
                    r1, r2 = unpk(plsc.load_gather(r_v, [rows, off_r + dv]))
                    t1, t2 = unpk(plsc.load_gather(t_v, [rows, off_t + dv]))
                    q1, q2 = unpk(plsc.load_gather(qr_v,
                                                   [rows, off_qr + dv]))
                    e1, e2 = unpk(plsc.load_gather(qe_v,
                                                   [rows, off_qe + dv]))
                    s1 = h1 + r1 - t1 + q1 - e1
                    s2 = h2 + r2 - t2 + q2 - e2
                    return acc + jnp.abs(s1) + jnp.abs(s2)

                dist = lax.fori_loop(0, _D // 2, comb_body, zero, unroll=8)
                sc_v[pl.ds(c * _CHUNK + g * 16, 16)] = dist
                return gcarry

            lax.fori_loop(0, _NGRP, group_body, 0)

        issue(0, 0)

        def pair_body(i, carry):
            c0 = 2 * i
            drain(0)
            issue(c0 + 1, 1)
            compute(c0, 0)
            drain(1)
            issue(lax.min(c0 + 2, _NCHUNK - 1), 0)
            compute(c0 + 1, 1)
            return carry

        lax.fori_loop(0, _NCHUNK // 2, pair_body, 0)
        drain(0)

        def loss_body(i, pacc):
            pv = sc_v[pl.ds(i * 16, 16)]
            nv = sc_v[pl.ds(_HALF_W + i * 16, 16)]
            return pacc + jnp.maximum(pv - nv + _MARGIN, 0.0)

        par_v[...] = lax.fori_loop(0, _HALF_W // 16, loss_body, zero,
                                   unroll=4)
        pltpu.sync_copy(sc_v.at[pl.ds(0, _HALF_W)],
                        scores_hbm.at[pl.ds(w * _HALF_W, _HALF_W)])
        pltpu.sync_copy(sc_v.at[pl.ds(_HALF_W, _HALF_W)],
                        scores_hbm.at[pl.ds(_B + w * _HALF_W, _HALF_W)])
        pltpu.sync_copy(par_v, part_hbm.at[w])

    return trans_e(ent2, rel2, idx_blocks)


def kernel(entity_emb, relation_emb, batch_positives, batch_negatives):
    ent2 = _pack_table(entity_emb, normalize=True)
    rel2 = _pack_table(relation_emb, normalize=False)
    pos = batch_positives.reshape(_NW, _NCHUNK // 2, _CHUNK, 5)
    neg = batch_negatives.reshape(_NW, _NCHUNK // 2, _CHUNK, 5)
    idx_blocks = jnp.concatenate(
        [pos.transpose(0, 1, 3, 2), neg.transpose(0, 1, 3, 2)], axis=1)
    scores, partials = _sc_call(ent2, rel2, idx_blocks)
    loss = jnp.sum(partials) / _B
    return scores[:_B], scores[_B:], loss
